# baseline (speedup 1.0000x reference)
import functools

import jax
import jax.numpy as jnp
from jax import lax
from jax.experimental import pallas as pl
from jax.experimental.pallas import tpu as pltpu
from jax.experimental.pallas import tpu_sc as plsc

N_NODES = 10000
NODE_DIM = 128
EDGE_DIM = 16
HIDDEN_DIM = 256

_NC = 2
_NS = 16
_NW = _NC * _NS

_GCH = 128


def _sc_gather(x, i_idx, j_idx):
    E = i_idx.shape[0]
    D = x.shape[1]
    dt = x.dtype
    n_chunks = E // _GCH
    per_w = (n_chunks + _NW - 1) // _NW
    mesh = plsc.VectorSubcoreMesh(core_axis_name="c", subcore_axis_name="s")

    @functools.partial(
        pl.kernel,
        out_type=(jax.ShapeDtypeStruct((E, D), dt),
                  jax.ShapeDtypeStruct((E, D), dt)),
        mesh=mesh,
        scratch_types=[
            pltpu.VMEM((2, _GCH), jnp.int32),
            pltpu.VMEM((2, _GCH), jnp.int32),
            pltpu.VMEM((2, _GCH, D), dt),
            pltpu.VMEM((2, _GCH, D), dt),
            pltpu.SemaphoreType.DMA,
            pltpu.SemaphoreType.DMA,
            pltpu.SemaphoreType.DMA((2,)),
        ],
    )
    def k(x_hbm, ii_hbm, jj_hbm, xi_hbm, xj_hbm, ib, jb, bi, bj,
          sem_idx, sem_g, sem_s):
        wid = lax.axis_index("s") * _NC + lax.axis_index("c")
        nloc = (n_chunks - wid + _NW - 1) // _NW

        def base(t):
            return (wid + _NW * t) * _GCH

        def issue_idx(t, p):
            pltpu.async_copy(ii_hbm.at[pl.ds(base(t), _GCH)], ib.at[p], sem_idx)
            pltpu.async_copy(jj_hbm.at[pl.ds(base(t), _GCH)], jb.at[p], sem_idx)

        def wait_idx(t, p):
            pltpu.make_async_copy(ii_hbm.at[pl.ds(base(t), _GCH)], ib.at[p],
                                  sem_idx).wait()
            pltpu.make_async_copy(jj_hbm.at[pl.ds(base(t), _GCH)], jb.at[p],
                                  sem_idx).wait()

        def issue_gather(p):
            pltpu.async_copy(x_hbm.at[ib.at[p]], bi.at[p], sem_g)
            pltpu.async_copy(x_hbm.at[jb.at[p]], bj.at[p], sem_g)

        def wait_gather(p):
            pltpu.make_async_copy(x_hbm.at[ib.at[p]], bi.at[p], sem_g).wait()
            pltpu.make_async_copy(x_hbm.at[jb.at[p]], bj.at[p], sem_g).wait()

        def issue_store(t, p):
            pltpu.async_copy(bi.at[p], xi_hbm.at[pl.ds(base(t), _GCH)],
                             sem_s.at[p])
            pltpu.async_copy(bj.at[p], xj_hbm.at[pl.ds(base(t), _GCH)],
                             sem_s.at[p])

        def wait_store(t, p):
            pltpu.make_async_copy(bi.at[p], xi_hbm.at[pl.ds(base(t), _GCH)],
                                  sem_s.at[p]).wait()
            pltpu.make_async_copy(bj.at[p], xj_hbm.at[pl.ds(base(t), _GCH)],
                                  sem_s.at[p]).wait()

        pltpu.sync_copy(ii_hbm.at[pl.ds(base(0), _GCH)], ib.at[0])
        pltpu.sync_copy(jj_hbm.at[pl.ds(base(0), _GCH)], jb.at[0])
        issue_gather(0)

        @pl.when(1 < nloc)
        def _():
            issue_idx(1, 1)

        @pl.loop(0, per_w)
        def _(t):
            @pl.when(t < nloc)
            def _():
                p = t & 1
                wait_gather(p)
                issue_store(t, p)

                @pl.when(t + 1 < nloc)
                def _():
                    wait_idx(t + 1, 1 - p)

                    @pl.when(t >= 1)
                    def _():
                        wait_store(t - 1, 1 - p)

                    issue_gather(1 - p)

                @pl.when(t + 2 < nloc)
                def _():
                    issue_idx(t + 2, p)

        pe = (nloc - 1) & 1
        wait_store(nloc - 1, pe)

        @pl.when(nloc > 1)
        def _():
            wait_store(nloc - 2, 1 - pe)

    return k(x, i_idx, j_idx)


def _sc_scatter_sum(ms, i_idx):
    K = len(ms)
    _, Ck, Fh = ms[0].shape
    E = i_idx.shape[0]
    n_chunks = Ck // _GCH
    per_s = (n_chunks + _NS - 1) // _NS
    RCH = 80
    n_rchunks = N_NODES // RCH
    per_s_rows = (n_rchunks + _NS - 1) // _NS
    mesh = plsc.VectorSubcoreMesh(core_axis_name="c", subcore_axis_name="s")

    @functools.partial(
        pl.kernel,
        out_type=jax.ShapeDtypeStruct((_NC, N_NODES, Fh), jnp.float32),
        mesh=mesh,
        scratch_types=[
            pltpu.VMEM((2, _GCH), jnp.int32),
            pltpu.VMEM((2, _GCH, Fh), jnp.float32),
            pltpu.VMEM_SHARED((N_NODES, Fh), jnp.float32),
            pltpu.SemaphoreType.DMA,
            pltpu.SemaphoreType.DMA((2,)),
        ],
    )
    def k(*args):
        m_hbms = args[:K]
        ii_hbm, agg_hbm, ib, mb, acc, sem_m, sem_a = args[K:]
        c = lax.axis_index("c")
        s = lax.axis_index("s")
        nloc = (n_chunks - s + _NS - 1) // _NS

        @pl.loop(0, RCH)
        def _(r):
            @pl.loop(0, Fh, step=16)
            def _(cc):
                mb[0, r, pl.ds(cc, 16)] = jnp.zeros((16,), jnp.float32)

        @pl.loop(0, per_s_rows)
        def _(kk):
            q = s + _NS * kk

            @pl.when(q < n_rchunks)
            def _():
                pltpu.async_copy(mb.at[0, pl.ds(0, RCH)],
                                 acc.at[pl.ds(q * RCH, RCH)], sem_m)

        @pl.loop(0, per_s_rows)
        def _(kk):
            q = s + _NS * kk

            @pl.when(q < n_rchunks)
            def _():
                pltpu.make_async_copy(mb.at[0, pl.ds(0, RCH)],
                                      acc.at[pl.ds(q * RCH, RCH)],
                                      sem_m).wait()

        plsc.subcore_barrier()

        def wait_add(p):
            pltpu.make_async_copy(mb.at[p], acc.at[ib.at[p]],
                                  sem_a.at[p]).wait()

        for ki in range(K):
            m_hbm = m_hbms[ki]
            ebase = ki * Ck

            def base(t, ebase=ebase):
                return ebase + (s + _NS * t) * _GCH

            def mbase(t):
                return (s + _NS * t) * _GCH

            def issue_load(t, p):
                pltpu.async_copy(ii_hbm.at[pl.ds(base(t), _GCH)], ib.at[p],
                                 sem_m)
                pltpu.async_copy(m_hbm.at[c, pl.ds(mbase(t), _GCH)], mb.at[p],
                                 sem_m)

            def wait_load(t, p):
                pltpu.make_async_copy(ii_hbm.at[pl.ds(base(t), _GCH)],
                                      ib.at[p], sem_m).wait()
                pltpu.make_async_copy(m_hbm.at[c, pl.ds(mbase(t), _GCH)],
                                      mb.at[p], sem_m).wait()

            issue_load(0, 0)

            @pl.loop(0, per_s)
            def _(t):
                @pl.when(t < nloc)
                def _():
                    p = t & 1
                    wait_load(t, p)
                    pltpu.async_copy(mb.at[p], acc.at[ib.at[p]], sem_a.at[p],
                                     add=True)

                    @pl.when(t + 1 < nloc)
                    def _():
                        @pl.when(t >= 1)
                        def _():
                            wait_add(1 - p)

                        issue_load(t + 1, 1 - p)

            wait_add((nloc - 1) & 1)

            @pl.when(nloc > 1)
            def _():
                wait_add(1 - ((nloc - 1) & 1))

        plsc.subcore_barrier()

        @pl.loop(0, per_s_rows)
        def _(kk):
            q = s + _NS * kk

            @pl.when(q < n_rchunks)
            def _():
                pltpu.async_copy(acc.at[pl.ds(q * RCH, RCH)],
                                 agg_hbm.at[c, pl.ds(q * RCH, RCH)], sem_m)

        @pl.loop(0, per_s_rows)
        def _(kk):
            q = s + _NS * kk

            @pl.when(q < n_rchunks)
            def _():
                pltpu.make_async_copy(acc.at[pl.ds(q * RCH, RCH)],
                                      agg_hbm.at[c, pl.ds(q * RCH, RCH)],
                                      sem_m).wait()

    return k(*ms, i_idx)


def _tc_edge_mlp(xi, xj, ea, w1a, w1b, w1c, b1, w2, b2):
    E, Dp = xi.shape
    H = w2.shape[0]
    Fh = H // _NC
    BE = 1000
    grid = (E // BE,)

    def body(xi_ref, xj_ref, ea_ref, w1a_ref, w1b_ref, w1c_ref, b1_ref,
             w2_ref, b2_ref, m_ref):
        bf = w2_ref.dtype
        h = jnp.dot(xi_ref[...].astype(bf), w1a_ref[...],
                    preferred_element_type=jnp.float32)
        h += jnp.dot(xj_ref[...].astype(bf), w1b_ref[...],
                     preferred_element_type=jnp.float32)
        h += jnp.dot(ea_ref[...], w1c_ref[...], preferred_element_type=jnp.float32)
        h = jnp.maximum(h + b1_ref[...], 0.0).astype(w2_ref.dtype)
        m = jnp.dot(h, w2_ref[...], preferred_element_type=jnp.float32)
        m = jnp.maximum(m + b2_ref[...], 0.0)
        m_ref[0] = m[:, :Fh]
        m_ref[1] = m[:, Fh:]

    fixed = lambda shape: pl.BlockSpec(shape, lambda i: (0,) * len(shape))
    return pl.pallas_call(
        body,
        grid=grid,
        in_specs=[
            pl.BlockSpec((BE, Dp), lambda i: (i, 0)),
            pl.BlockSpec((BE, Dp), lambda i: (i, 0)),
            pl.BlockSpec((BE, EDGE_DIM), lambda i: (i, 0)),
            fixed(w1a.shape),
            fixed(w1b.shape),
            fixed(w1c.shape),
            fixed((1, H)),
            fixed(w2.shape),
            fixed((1, H)),
        ],
        out_specs=pl.BlockSpec((_NC, BE, Fh), lambda i: (0, i, 0)),
        out_shape=jax.ShapeDtypeStruct((_NC, E, Fh), jnp.float32),
    )(xi, xj, ea, w1a, w1b, w1c, b1, w2, b2)


def _tc_node_mlp(x, aggs, w3a, w3b0, w3b1, b3, gamma, beta):
    N, D = x.shape
    K = len(aggs)
    Fh = aggs[0].shape[2]
    BN = 1000
    grid = (N // BN,)

    def body(x_ref, *rest):
        agg_refs = rest[:K]
        w3a_ref, w3b0_ref, w3b1_ref, b3_ref, g_ref, be_ref, o_ref = rest[K:]
        a0 = agg_refs[0][0]
        a1 = agg_refs[0][1]
        for r in agg_refs[1:]:
            a0 += r[0]
            a1 += r[1]
        xv = x_ref[...]
        out = jnp.dot(xv, w3a_ref[...], preferred_element_type=jnp.float32)
        out += jnp.dot(a0, w3b0_ref[...], preferred_element_type=jnp.float32)
        out += jnp.dot(a1, w3b1_ref[...], preferred_element_type=jnp.float32)
        out = jnp.maximum(out + b3_ref[...], 0.0) + xv
        mu = jnp.mean(out, axis=-1, keepdims=True)
        d = out - mu
        var = jnp.mean(d * d, axis=-1, keepdims=True)
        o_ref[...] = d * lax.rsqrt(var + 1e-5) * g_ref[...] + be_ref[...]

    fixed = lambda shape: pl.BlockSpec(shape, lambda i: (0,) * len(shape))
    return pl.pallas_call(
        body,
        grid=grid,
        in_specs=[
            pl.BlockSpec((BN, D), lambda i: (i, 0)),
        ] + [
            pl.BlockSpec((_NC, BN, Fh), lambda i: (0, i, 0)) for _ in range(K)
        ] + [
            fixed(w3a.shape),
            fixed(w3b0.shape),
            fixed(w3b1.shape),
            fixed((1, D)),
            fixed((1, D)),
            fixed((1, D)),
        ],
        out_specs=pl.BlockSpec((BN, D), lambda i: (i, 0)),
        out_shape=jax.ShapeDtypeStruct((N, D), jnp.float32),
    )(x, *aggs, w3a, w3b0, w3b1, b3, gamma, beta)


def kernel(x, edge_index, edge_attr, W1, b1, W2, b2, W3, b3, gamma, beta):
    i_idx = edge_index[0].astype(jnp.int32)
    j_idx = edge_index[1].astype(jnp.int32)
    D = x.shape[1]
    H = W2.shape[0]
    bf = jnp.bfloat16

    w1a = W1[:D].astype(bf)
    w1b = W1[D:2 * D].astype(bf)
    w1c = W1[2 * D:].astype(bf)
    b1r = b1.reshape(1, H)
    w2 = W2.astype(bf)
    b2r = b2.reshape(1, H)
    ea = edge_attr.astype(bf)

    K = 2
    E = i_idx.shape[0]
    Ck = E // K
    aggs = []
    for k in range(K):
        sl = slice(k * Ck, (k + 1) * Ck)
        xi, xj = _sc_gather(x, i_idx[sl], j_idx[sl])
        m = _tc_edge_mlp(xi, xj, ea[sl], w1a, w1b, w1c, b1r, w2, b2r)
        aggs.append(_sc_scatter_sum([m], i_idx[sl]))

    Fh = H // _NC
    out = _tc_node_mlp(
        x, aggs, W3[:D], W3[D:D + Fh], W3[D + Fh:], b3.reshape(1, D),
        gamma.reshape(1, D), beta.reshape(1, D),
    )
    return out


# speedup vs baseline: 1.0535x; 1.0535x over previous
import functools

import jax
import jax.numpy as jnp
from jax import lax
from jax.experimental import pallas as pl
from jax.experimental.pallas import tpu as pltpu
from jax.experimental.pallas import tpu_sc as plsc

N_NODES = 10000
NODE_DIM = 128
EDGE_DIM = 16
HIDDEN_DIM = 256

_NC = 2
_NS = 16
_NW = _NC * _NS

_GCH = 128


def _sc_gather(x, i_idx, j_idx):
    E = i_idx.shape[0]
    D = x.shape[1]
    dt = x.dtype
    n_chunks = E // _GCH
    per_w = (n_chunks + _NW - 1) // _NW
    mesh = plsc.VectorSubcoreMesh(core_axis_name="c", subcore_axis_name="s")

    @functools.partial(
        pl.kernel,
        out_type=(jax.ShapeDtypeStruct((E, D), dt),
                  jax.ShapeDtypeStruct((E, D), dt)),
        mesh=mesh,
        scratch_types=[
            pltpu.VMEM((2, _GCH), jnp.int32),
            pltpu.VMEM((2, _GCH), jnp.int32),
            pltpu.VMEM((2, _GCH, D), dt),
            pltpu.VMEM((2, _GCH, D), dt),
            pltpu.SemaphoreType.DMA,
            pltpu.SemaphoreType.DMA,
            pltpu.SemaphoreType.DMA((2,)),
        ],
    )
    def k(x_hbm, ii_hbm, jj_hbm, xi_hbm, xj_hbm, ib, jb, bi, bj,
          sem_idx, sem_g, sem_s):
        wid = lax.axis_index("s") * _NC + lax.axis_index("c")
        nloc = (n_chunks - wid + _NW - 1) // _NW

        def base(t):
            return (wid + _NW * t) * _GCH

        def issue_idx(t, p):
            pltpu.async_copy(ii_hbm.at[pl.ds(base(t), _GCH)], ib.at[p], sem_idx)
            pltpu.async_copy(jj_hbm.at[pl.ds(base(t), _GCH)], jb.at[p], sem_idx)

        def wait_idx(t, p):
            pltpu.make_async_copy(ii_hbm.at[pl.ds(base(t), _GCH)], ib.at[p],
                                  sem_idx).wait()
            pltpu.make_async_copy(jj_hbm.at[pl.ds(base(t), _GCH)], jb.at[p],
                                  sem_idx).wait()

        def issue_gather(p):
            pltpu.async_copy(x_hbm.at[ib.at[p]], bi.at[p], sem_g)
            pltpu.async_copy(x_hbm.at[jb.at[p]], bj.at[p], sem_g)

        def wait_gather(p):
            pltpu.make_async_copy(x_hbm.at[ib.at[p]], bi.at[p], sem_g).wait()
            pltpu.make_async_copy(x_hbm.at[jb.at[p]], bj.at[p], sem_g).wait()

        def issue_store(t, p):
            pltpu.async_copy(bi.at[p], xi_hbm.at[pl.ds(base(t), _GCH)],
                             sem_s.at[p])
            pltpu.async_copy(bj.at[p], xj_hbm.at[pl.ds(base(t), _GCH)],
                             sem_s.at[p])

        def wait_store(t, p):
            pltpu.make_async_copy(bi.at[p], xi_hbm.at[pl.ds(base(t), _GCH)],
                                  sem_s.at[p]).wait()
            pltpu.make_async_copy(bj.at[p], xj_hbm.at[pl.ds(base(t), _GCH)],
                                  sem_s.at[p]).wait()

        pltpu.sync_copy(ii_hbm.at[pl.ds(base(0), _GCH)], ib.at[0])
        pltpu.sync_copy(jj_hbm.at[pl.ds(base(0), _GCH)], jb.at[0])
        issue_gather(0)

        @pl.when(1 < nloc)
        def _():
            issue_idx(1, 1)

        @pl.loop(0, per_w)
        def _(t):
            @pl.when(t < nloc)
            def _():
                p = t & 1
                wait_gather(p)
                issue_store(t, p)

                @pl.when(t + 1 < nloc)
                def _():
                    wait_idx(t + 1, 1 - p)

                    @pl.when(t >= 1)
                    def _():
                        wait_store(t - 1, 1 - p)

                    issue_gather(1 - p)

                @pl.when(t + 2 < nloc)
                def _():
                    issue_idx(t + 2, p)

        pe = (nloc - 1) & 1
        wait_store(nloc - 1, pe)

        @pl.when(nloc > 1)
        def _():
            wait_store(nloc - 2, 1 - pe)

    return k(x, i_idx, j_idx)


def _sc_scatter_sum(ms, i_idx):
    K = len(ms)
    _, Ck, Fh = ms[0].shape
    E = i_idx.shape[0]
    n_chunks = Ck // _GCH
    per_s = (n_chunks + _NS - 1) // _NS
    RCH = 80
    n_rchunks = N_NODES // RCH
    per_s_rows = (n_rchunks + _NS - 1) // _NS
    mesh = plsc.VectorSubcoreMesh(core_axis_name="c", subcore_axis_name="s")

    @functools.partial(
        pl.kernel,
        out_type=jax.ShapeDtypeStruct((_NC, N_NODES, Fh), jnp.float32),
        mesh=mesh,
        scratch_types=[
            pltpu.VMEM((3, _GCH), jnp.int32),
            pltpu.VMEM((3, _GCH, Fh), jnp.float32),
            pltpu.VMEM_SHARED((N_NODES, Fh), jnp.float32),
            pltpu.SemaphoreType.DMA,
            pltpu.SemaphoreType.DMA((3,)),
        ],
    )
    def k(*args):
        m_hbms = args[:K]
        ii_hbm, agg_hbm, ib, mb, acc, sem_m, sem_a = args[K:]
        c = lax.axis_index("c")
        s = lax.axis_index("s")
        nloc = (n_chunks - s + _NS - 1) // _NS

        @pl.loop(0, RCH)
        def _(r):
            @pl.loop(0, Fh, step=16)
            def _(cc):
                mb[0, r, pl.ds(cc, 16)] = jnp.zeros((16,), jnp.float32)

        @pl.loop(0, per_s_rows)
        def _(kk):
            q = s + _NS * kk

            @pl.when(q < n_rchunks)
            def _():
                pltpu.async_copy(mb.at[0, pl.ds(0, RCH)],
                                 acc.at[pl.ds(q * RCH, RCH)], sem_m)

        @pl.loop(0, per_s_rows)
        def _(kk):
            q = s + _NS * kk

            @pl.when(q < n_rchunks)
            def _():
                pltpu.make_async_copy(mb.at[0, pl.ds(0, RCH)],
                                      acc.at[pl.ds(q * RCH, RCH)],
                                      sem_m).wait()

        plsc.subcore_barrier()

        def wait_add(p):
            pltpu.make_async_copy(mb.at[p], acc.at[ib.at[p]],
                                  sem_a.at[p]).wait()

        for ki in range(K):
            m_hbm = m_hbms[ki]
            ebase = ki * Ck

            def base(t, ebase=ebase):
                return ebase + (s + _NS * t) * _GCH

            def mbase(t):
                return (s + _NS * t) * _GCH

            def issue_load(t, p):
                pltpu.async_copy(ii_hbm.at[pl.ds(base(t), _GCH)], ib.at[p],
                                 sem_m)
                pltpu.async_copy(m_hbm.at[c, pl.ds(mbase(t), _GCH)], mb.at[p],
                                 sem_m)

            def wait_load(t, p):
                pltpu.make_async_copy(ii_hbm.at[pl.ds(base(t), _GCH)],
                                      ib.at[p], sem_m).wait()
                pltpu.make_async_copy(m_hbm.at[c, pl.ds(mbase(t), _GCH)],
                                      mb.at[p], sem_m).wait()

            issue_load(0, 0)

            @pl.when(1 < nloc)
            def _():
                issue_load(1, 1)

            @pl.loop(0, per_s)
            def _(t):
                @pl.when(t < nloc)
                def _():
                    p = t % 3
                    wait_load(t, p)
                    pltpu.async_copy(mb.at[p], acc.at[ib.at[p]], sem_a.at[p],
                                     add=True)

                    @pl.when(t + 2 < nloc)
                    def _():
                        q = (t + 2) % 3

                        @pl.when(t >= 1)
                        def _():
                            wait_add(q)

                        issue_load(t + 2, q)

            wait_add((nloc - 1) % 3)

            @pl.when(nloc > 1)
            def _():
                wait_add((nloc - 2) % 3)

            @pl.when(nloc > 2)
            def _():
                wait_add((nloc - 3) % 3)

        plsc.subcore_barrier()

        @pl.loop(0, per_s_rows)
        def _(kk):
            q = s + _NS * kk

            @pl.when(q < n_rchunks)
            def _():
                pltpu.async_copy(acc.at[pl.ds(q * RCH, RCH)],
                                 agg_hbm.at[c, pl.ds(q * RCH, RCH)], sem_m)

        @pl.loop(0, per_s_rows)
        def _(kk):
            q = s + _NS * kk

            @pl.when(q < n_rchunks)
            def _():
                pltpu.make_async_copy(acc.at[pl.ds(q * RCH, RCH)],
                                      agg_hbm.at[c, pl.ds(q * RCH, RCH)],
                                      sem_m).wait()

    return k(*ms, i_idx)


def _tc_edge_mlp(xi, xj, ea, w1a, w1b, w1c, b1, w2, b2):
    E, Dp = xi.shape
    H = w2.shape[0]
    Fh = H // _NC
    BE = 1000
    grid = (E // BE,)

    def body(xi_ref, xj_ref, ea_ref, w1a_ref, w1b_ref, w1c_ref, b1_ref,
             w2_ref, b2_ref, m_ref):
        bf = w2_ref.dtype
        h = jnp.dot(xi_ref[...].astype(bf), w1a_ref[...],
                    preferred_element_type=jnp.float32)
        h += jnp.dot(xj_ref[...].astype(bf), w1b_ref[...],
                     preferred_element_type=jnp.float32)
        h += jnp.dot(ea_ref[...], w1c_ref[...], preferred_element_type=jnp.float32)
        h = jnp.maximum(h + b1_ref[...], 0.0).astype(w2_ref.dtype)
        m = jnp.dot(h, w2_ref[...], preferred_element_type=jnp.float32)
        m = jnp.maximum(m + b2_ref[...], 0.0)
        m_ref[0] = m[:, :Fh]
        m_ref[1] = m[:, Fh:]

    fixed = lambda shape: pl.BlockSpec(shape, lambda i: (0,) * len(shape))
    return pl.pallas_call(
        body,
        grid=grid,
        in_specs=[
            pl.BlockSpec((BE, Dp), lambda i: (i, 0)),
            pl.BlockSpec((BE, Dp), lambda i: (i, 0)),
            pl.BlockSpec((BE, EDGE_DIM), lambda i: (i, 0)),
            fixed(w1a.shape),
            fixed(w1b.shape),
            fixed(w1c.shape),
            fixed((1, H)),
            fixed(w2.shape),
            fixed((1, H)),
        ],
        out_specs=pl.BlockSpec((_NC, BE, Fh), lambda i: (0, i, 0)),
        out_shape=jax.ShapeDtypeStruct((_NC, E, Fh), jnp.float32),
    )(xi, xj, ea, w1a, w1b, w1c, b1, w2, b2)


def _tc_node_mlp(x, aggs, w3a, w3b0, w3b1, b3, gamma, beta):
    N, D = x.shape
    K = len(aggs)
    Fh = aggs[0].shape[2]
    BN = 1000
    grid = (N // BN,)

    def body(x_ref, *rest):
        agg_refs = rest[:K]
        w3a_ref, w3b0_ref, w3b1_ref, b3_ref, g_ref, be_ref, o_ref = rest[K:]
        a0 = agg_refs[0][0]
        a1 = agg_refs[0][1]
        for r in agg_refs[1:]:
            a0 += r[0]
            a1 += r[1]
        xv = x_ref[...]
        out = jnp.dot(xv, w3a_ref[...], preferred_element_type=jnp.float32)
        out += jnp.dot(a0, w3b0_ref[...], preferred_element_type=jnp.float32)
        out += jnp.dot(a1, w3b1_ref[...], preferred_element_type=jnp.float32)
        out = jnp.maximum(out + b3_ref[...], 0.0) + xv
        mu = jnp.mean(out, axis=-1, keepdims=True)
        d = out - mu
        var = jnp.mean(d * d, axis=-1, keepdims=True)
        o_ref[...] = d * lax.rsqrt(var + 1e-5) * g_ref[...] + be_ref[...]

    fixed = lambda shape: pl.BlockSpec(shape, lambda i: (0,) * len(shape))
    return pl.pallas_call(
        body,
        grid=grid,
        in_specs=[
            pl.BlockSpec((BN, D), lambda i: (i, 0)),
        ] + [
            pl.BlockSpec((_NC, BN, Fh), lambda i: (0, i, 0)) for _ in range(K)
        ] + [
            fixed(w3a.shape),
            fixed(w3b0.shape),
            fixed(w3b1.shape),
            fixed((1, D)),
            fixed((1, D)),
            fixed((1, D)),
        ],
        out_specs=pl.BlockSpec((BN, D), lambda i: (i, 0)),
        out_shape=jax.ShapeDtypeStruct((N, D), jnp.float32),
    )(x, *aggs, w3a, w3b0, w3b1, b3, gamma, beta)


def kernel(x, edge_index, edge_attr, W1, b1, W2, b2, W3, b3, gamma, beta):
    i_idx = edge_index[0].astype(jnp.int32)
    j_idx = edge_index[1].astype(jnp.int32)
    D = x.shape[1]
    H = W2.shape[0]
    bf = jnp.bfloat16

    w1a = W1[:D].astype(bf)
    w1b = W1[D:2 * D].astype(bf)
    w1c = W1[2 * D:].astype(bf)
    b1r = b1.reshape(1, H)
    w2 = W2.astype(bf)
    b2r = b2.reshape(1, H)
    ea = edge_attr.astype(bf)

    K = 4
    E = i_idx.shape[0]
    Ck = E // K
    aggs = []
    for k in range(K):
        sl = slice(k * Ck, (k + 1) * Ck)
        xi, xj = _sc_gather(x, i_idx[sl], j_idx[sl])
        m = _tc_edge_mlp(xi, xj, ea[sl], w1a, w1b, w1c, b1r, w2, b2r)
        aggs.append(_sc_scatter_sum([m], i_idx[sl]))

    Fh = H // _NC
    out = _tc_node_mlp(
        x, aggs, W3[:D], W3[D:D + Fh], W3[D + Fh:],
        b3.reshape(1, D), gamma.reshape(1, D), beta.reshape(1, D),
    )
    return out
